# baseline (speedup 1.0000x reference)
import functools

import jax
import jax.numpy as jnp
from jax import lax
from jax.experimental import pallas as pl
from jax.experimental.pallas import tpu as pltpu
from jax.experimental.pallas import tpu_sc as plsc

H = 128
N_NODES = 10000
M_MSG = 160000
K_NEI = 4
B_TREE = 256

_NC = 2
_NS = 16
_NW = _NC * _NS

_F32 = jnp.float32
_I32 = jnp.int32


def _wid():
    return lax.axis_index("s") * _NC + lax.axis_index("c")


def _sc_mesh():
    return plsc.VectorSubcoreMesh(core_axis_name="c", subcore_axis_name="s")


def _make_row_gather(total, d, chunk, active, n_table_rows):
    per_w = total // active
    n_chunks = per_w // chunk
    assert per_w * active == total and n_chunks * chunk == per_w
    assert chunk % 8 == 0 and per_w % 8 == 0

    @functools.partial(
        pl.kernel,
        mesh=_sc_mesh(),
        out_type=jax.ShapeDtypeStruct((total, d), _F32),
        scratch_types=[
            pltpu.VMEM((chunk,), _I32),
            pltpu.VMEM((chunk,), _I32),
            pltpu.VMEM((chunk, d), _F32),
            pltpu.VMEM((chunk, d), _F32),
            pltpu.SemaphoreType.DMA,
        ],
    )
    def gather_k(table_hbm, idx_hbm, out_hbm, idx_a, idx_b, rows_a, rows_b,
                 sem):
        w = _wid()
        idx_bufs = (idx_a, idx_b)
        row_bufs = (rows_a, rows_b)

        @pl.when(w < active)
        def _():
            def issue(j, b):
                base = w * per_w + j * chunk
                pltpu.sync_copy(idx_hbm.at[pl.ds(base, chunk)], idx_bufs[b])
                pltpu.make_async_copy(
                    table_hbm.at[idx_bufs[b]], row_bufs[b], sem).start()

            issue(0, 0)

            def outer(j0, carry):
                for b in range(2):
                    j = 2 * j0 + b

                    @pl.when(j < n_chunks)
                    def _():
                        @pl.when(j + 1 < n_chunks)
                        def _():
                            issue(j + 1, 1 - b)

                        pltpu.make_async_copy(
                            table_hbm.at[idx_bufs[b]], row_bufs[b],
                            sem).wait()
                        base = w * per_w + j * chunk
                        pltpu.sync_copy(row_bufs[b],
                                        out_hbm.at[pl.ds(base, chunk)])

                return carry

            lax.fori_loop(0, (n_chunks + 1) // 2, outer, 0)

    return gather_k


def _make_final_gather():
    mb = (K_NEI * B_TREE) // _NW
    fb = B_TREE // _NS

    @functools.partial(
        pl.kernel,
        mesh=_sc_mesh(),
        out_type=(
            jax.ShapeDtypeStruct((K_NEI * B_TREE, H), _F32),
            jax.ShapeDtypeStruct((B_TREE, H), _F32),
        ),
        scratch_types=[
            pltpu.VMEM((mb,), _I32),
            pltpu.VMEM((mb, H), _F32),
            pltpu.VMEM((fb,), _I32),
            pltpu.VMEM((fb, H), _F32),
            pltpu.SemaphoreType.DMA,
        ],
    )
    def final_k(mess_hbm, femb_hbm, ngf_hbm, sidx_hbm, mess_s_hbm, femb_s_hbm,
                i1_v, r1_v, i2_v, r2_v, sem):
        w = _wid()
        base = w * mb
        pltpu.sync_copy(ngf_hbm.at[pl.ds(base, mb)], i1_v)
        pltpu.async_copy(mess_hbm.at[i1_v], r1_v, sem).wait()
        pltpu.sync_copy(r1_v, mess_s_hbm.at[pl.ds(base, mb)])

        @pl.when(w < _NS)
        def _():
            fbase = w * fb
            pltpu.sync_copy(sidx_hbm.at[pl.ds(fbase, fb)], i2_v)
            pltpu.async_copy(femb_hbm.at[i2_v], r2_v, sem).wait()
            pltpu.sync_copy(r2_v, femb_s_hbm.at[pl.ds(fbase, fb)])

    return final_k


def _pre_body(femb_ref, wcat_ref, bcat_ref, pnode_ref, h1n_ref):
    e = femb_ref[...]
    p = jnp.dot(e, wcat_ref[...], preferred_element_type=_F32) + bcat_ref[...]
    pnode_ref[...] = p
    h1n_ref[...] = jax.nn.sigmoid(p[:, :H]) * jnp.tanh(p[:, 2 * H:])


def _precompute(femb, wcat, bcat):
    tn = 2000
    return pl.pallas_call(
        _pre_body,
        grid=(N_NODES // tn,),
        in_specs=[
            pl.BlockSpec((tn, H), lambda i: (i, 0)),
            pl.BlockSpec((H, 3 * H), lambda i: (0, 0)),
            pl.BlockSpec((1, 3 * H), lambda i: (0, 0)),
        ],
        out_specs=[
            pl.BlockSpec((tn, 3 * H), lambda i: (i, 0)),
            pl.BlockSpec((tn, H), lambda i: (i, 0)),
        ],
        out_shape=[
            jax.ShapeDtypeStruct((N_NODES, 3 * H), _F32),
            jax.ShapeDtypeStruct((N_NODES, H), _F32),
        ],
    )(femb, wcat, bcat)


_TM = 2000


def _depth_body(xcat_ref, hn_ref, wzb_ref, ur_ref, bur_ref, whb_ref, hout_ref):
    i = pl.program_id(0)
    xz = xcat_ref[:, :H]
    xr = xcat_ref[:, H:2 * H]
    xh = xcat_ref[:, 2 * H:]
    h0 = hn_ref[0]
    h1 = hn_ref[1]
    h2 = hn_ref[2]
    h3 = hn_ref[3]
    sum_h = (h0 + h1) + (h2 + h3)
    z = jax.nn.sigmoid(xz + jnp.dot(sum_h, wzb_ref[...],
                                    preferred_element_type=_F32))
    ur = ur_ref[...]
    bur = bur_ref[...]
    sg = jnp.zeros_like(sum_h)
    for hk in (h0, h1, h2, h3):
        rk = jax.nn.sigmoid(
            xr + jnp.dot(hk, ur, preferred_element_type=_F32) + bur)
        sg = sg + rk * hk
    pre = jnp.tanh(xh + jnp.dot(sg, whb_ref[...],
                                preferred_element_type=_F32))
    hnew = sum_h + z * (pre - sum_h)
    row = lax.broadcasted_iota(_I32, hnew.shape, 0) + i * _TM
    hout_ref[...] = jnp.where(row == 0, 0.0, hnew)


def _depth(xcat, hn, wzb, ur, bur, whb):
    return pl.pallas_call(
        _depth_body,
        grid=(M_MSG // _TM,),
        in_specs=[
            pl.BlockSpec((_TM, 3 * H), lambda i: (i, 0)),
            pl.BlockSpec((K_NEI, _TM, H), lambda i: (0, i, 0)),
            pl.BlockSpec((H, H), lambda i: (0, 0)),
            pl.BlockSpec((H, H), lambda i: (0, 0)),
            pl.BlockSpec((1, H), lambda i: (0, 0)),
            pl.BlockSpec((H, H), lambda i: (0, 0)),
        ],
        out_specs=pl.BlockSpec((_TM, H), lambda i: (i, 0)),
        out_shape=jax.ShapeDtypeStruct((M_MSG, H), _F32),
    )(xcat, hn, wzb, ur, bur, whb)


def _out_body(mess_s_ref, femb_s_ref, wot_ref, wob_ref, bo_ref, tree_ref):
    nsum = (mess_s_ref[0] + mess_s_ref[1]) + (mess_s_ref[2] + mess_s_ref[3])
    acc = jnp.dot(femb_s_ref[...], wot_ref[...], preferred_element_type=_F32)
    acc = acc + jnp.dot(nsum, wob_ref[...], preferred_element_type=_F32)
    tree_ref[...] = jax.nn.relu(acc + bo_ref[...])


def _out_stage(mess_s, femb_s, wot, wob, bo):
    return pl.pallas_call(
        _out_body,
        out_shape=jax.ShapeDtypeStruct((B_TREE, H), _F32),
    )(mess_s, femb_s, wot, wob, bo)


def kernel(fnode, fmess, node_graph, mess_graph, scope, embedding,
           W_z, b_z, W_r, U_r, b_Ur, W_h, b_h, W_o, b_o):
    fnode = fnode.astype(_I32)
    fmess = fmess.astype(_I32)

    mgt = mess_graph.T.reshape(-1)
    idx1 = jnp.where(mgt == 0, N_NODES, jnp.take(fmess, mgt))
    sidx = scope[:, 0]
    ngf = jnp.take(node_graph, sidx, axis=0).T.reshape(-1)

    wcat = jnp.concatenate([W_z[:H], W_r, W_h[:H]], axis=1)
    bcat = jnp.concatenate(
        [b_z, jnp.zeros((H,), _F32), b_h]).reshape(1, 3 * H)

    femb = _make_row_gather(N_NODES, H, 400, 25, 800)(embedding, fnode)
    pnode, h1node = _precompute(femb, wcat, bcat)
    h1e = jnp.concatenate([h1node, jnp.zeros((8, H), _F32)], axis=0)
    xcat = _make_row_gather(M_MSG, 3 * H, 40, _NW, N_NODES)(pnode, fmess)

    wzb = W_z[H:]
    whb = W_h[H:]
    bur = b_Ur.reshape(1, H)
    hn = _make_row_gather(K_NEI * M_MSG, H, 400, _NW, N_NODES + 8)(
        h1e, idx1).reshape(K_NEI, M_MSG, H)
    h = _depth(xcat, hn, wzb, U_r, bur, whb)
    hn = _make_row_gather(K_NEI * M_MSG, H, 400, _NW, M_MSG)(
        h, mgt).reshape(K_NEI, M_MSG, H)
    messages = _depth(xcat, hn, wzb, U_r, bur, whb)

    mess_s, femb_s = _make_final_gather()(messages, femb, ngf, sidx)
    tree_vecs = _out_stage(mess_s.reshape(K_NEI, B_TREE, H), femb_s,
                           W_o[:H], W_o[H:], b_o.reshape(1, H))
    return (tree_vecs, messages)


# speedup vs baseline: 7.2382x; 7.2382x over previous
import functools

import jax
import jax.numpy as jnp
from jax import lax
from jax.experimental import pallas as pl
from jax.experimental.pallas import tpu as pltpu
from jax.experimental.pallas import tpu_sc as plsc

H = 128
N_NODES = 10000
M_MSG = 160000
K_NEI = 4
B_TREE = 256

_NC = 2
_NS = 16
_NW = _NC * _NS

_F32 = jnp.float32
_I32 = jnp.int32


def _wid():
    return lax.axis_index("s") * _NC + lax.axis_index("c")


def _sc_mesh():
    return plsc.VectorSubcoreMesh(core_axis_name="c", subcore_axis_name="s")


def _make_row_gather(total, d, chunk, active, n_table_rows):
    per_w = total // active
    n_chunks = per_w // chunk
    assert per_w * active == total and n_chunks * chunk == per_w
    assert chunk % 8 == 0 and per_w % 8 == 0

    @functools.partial(
        pl.kernel,
        mesh=_sc_mesh(),
        out_type=jax.ShapeDtypeStruct((total, d), _F32),
        scratch_types=[
            pltpu.VMEM((chunk,), _I32),
            pltpu.VMEM((chunk,), _I32),
            pltpu.VMEM((chunk, d), _F32),
            pltpu.VMEM((chunk, d), _F32),
            pltpu.SemaphoreType.DMA,
            pltpu.SemaphoreType.DMA,
        ],
    )
    def gather_k(table_hbm, idx_hbm, out_hbm, idx_a, idx_b, rows_a, rows_b,
                 sem_a, sem_b):
        w = _wid()
        idx_bufs = (idx_a, idx_b)
        row_bufs = (rows_a, rows_b)
        sems = (sem_a, sem_b)

        @pl.when(w < active)
        def _():
            def issue(j, b):
                base = w * per_w + j * chunk
                pltpu.sync_copy(idx_hbm.at[pl.ds(base, chunk)], idx_bufs[b])
                pltpu.make_async_copy(
                    table_hbm.at[idx_bufs[b]], row_bufs[b], sems[b]).start()

            issue(0, 0)

            def outer(j0, carry):
                for b in range(2):
                    j = 2 * j0 + b

                    @pl.when(j < n_chunks)
                    def _():
                        @pl.when(j + 1 < n_chunks)
                        def _():
                            issue(j + 1, 1 - b)

                        pltpu.make_async_copy(
                            table_hbm.at[idx_bufs[b]], row_bufs[b],
                            sems[b]).wait()
                        base = w * per_w + j * chunk
                        pltpu.sync_copy(row_bufs[b],
                                        out_hbm.at[pl.ds(base, chunk)])

                return carry

            lax.fori_loop(0, (n_chunks + 1) // 2, outer, 0)

    return gather_k


def _make_xcat_gather(chunk):
    per_w = M_MSG // _NW
    n_chunks = per_w // chunk
    assert n_chunks * chunk == per_w and chunk % 8 == 0

    @functools.partial(
        pl.kernel,
        mesh=_sc_mesh(),
        out_type=(
            jax.ShapeDtypeStruct((M_MSG, 3 * H), _F32),
            jax.ShapeDtypeStruct((M_MSG, H), _F32),
        ),
        scratch_types=[
            pltpu.VMEM((chunk,), _I32),
            pltpu.VMEM((chunk,), _I32),
            pltpu.VMEM((chunk, 3 * H), _F32),
            pltpu.VMEM((chunk, 3 * H), _F32),
            pltpu.VMEM((chunk, H), _F32),
            pltpu.VMEM((chunk, H), _F32),
            pltpu.VMEM((1, H), _F32),
            pltpu.SemaphoreType.DMA,
            pltpu.SemaphoreType.DMA,
        ],
    )
    def xcat_k(pnode_hbm, h1node_hbm, fmess_hbm, xcat_hbm, h1_hbm,
               idx_a, idx_b, p_a, p_b, h_a, h_b, z_v, sem_a, sem_b):
        w = _wid()
        idx_bufs = (idx_a, idx_b)
        p_bufs = (p_a, p_b)
        h_bufs = (h_a, h_b)
        sems = (sem_a, sem_b)

        def issue(j, b):
            base = w * per_w + j * chunk
            pltpu.sync_copy(fmess_hbm.at[pl.ds(base, chunk)], idx_bufs[b])
            pltpu.make_async_copy(
                pnode_hbm.at[idx_bufs[b]], p_bufs[b], sems[b]).start()
            pltpu.make_async_copy(
                h1node_hbm.at[idx_bufs[b]], h_bufs[b], sems[b]).start()

        issue(0, 0)

        def outer(j0, carry):
            for b in range(2):
                j = 2 * j0 + b

                @pl.when(j < n_chunks)
                def _():
                    @pl.when(j + 1 < n_chunks)
                    def _():
                        issue(j + 1, 1 - b)

                    pltpu.make_async_copy(
                        pnode_hbm.at[idx_bufs[b]], p_bufs[b], sems[b]).wait()
                    pltpu.make_async_copy(
                        h1node_hbm.at[idx_bufs[b]], h_bufs[b],
                        sems[b]).wait()
                    base = w * per_w + j * chunk
                    pltpu.sync_copy(p_bufs[b], xcat_hbm.at[pl.ds(base, chunk)])
                    pltpu.sync_copy(h_bufs[b], h1_hbm.at[pl.ds(base, chunk)])

            return carry

        lax.fori_loop(0, (n_chunks + 1) // 2, outer, 0)

        @pl.when(w == 0)
        def _():
            for c in range(H // 16):
                z_v[0, pl.ds(c * 16, 16)] = jnp.zeros((16,), _F32)
            pltpu.sync_copy(z_v, h1_hbm.at[pl.ds(0, 1)])

    return xcat_k


def _make_final_gather():
    mb = (K_NEI * B_TREE) // _NW
    fb = B_TREE // _NS

    @functools.partial(
        pl.kernel,
        mesh=_sc_mesh(),
        out_type=(
            jax.ShapeDtypeStruct((K_NEI * B_TREE, H), _F32),
            jax.ShapeDtypeStruct((B_TREE, H), _F32),
        ),
        scratch_types=[
            pltpu.VMEM((mb,), _I32),
            pltpu.VMEM((mb, H), _F32),
            pltpu.VMEM((fb,), _I32),
            pltpu.VMEM((fb, H), _F32),
            pltpu.SemaphoreType.DMA,
        ],
    )
    def final_k(mess_hbm, femb_hbm, ngf_hbm, sidx_hbm, mess_s_hbm, femb_s_hbm,
                i1_v, r1_v, i2_v, r2_v, sem):
        w = _wid()
        base = w * mb
        pltpu.sync_copy(ngf_hbm.at[pl.ds(base, mb)], i1_v)
        pltpu.async_copy(mess_hbm.at[i1_v], r1_v, sem).wait()
        pltpu.sync_copy(r1_v, mess_s_hbm.at[pl.ds(base, mb)])

        @pl.when(w < _NS)
        def _():
            fbase = w * fb
            pltpu.sync_copy(sidx_hbm.at[pl.ds(fbase, fb)], i2_v)
            pltpu.async_copy(femb_hbm.at[i2_v], r2_v, sem).wait()
            pltpu.sync_copy(r2_v, femb_s_hbm.at[pl.ds(fbase, fb)])

    return final_k


def _pre_body(femb_ref, wcat_ref, bcat_ref, pnode_ref, h1n_ref):
    e = femb_ref[...]
    p = jnp.dot(e, wcat_ref[...], preferred_element_type=_F32) + bcat_ref[...]
    pnode_ref[...] = p
    h1n_ref[...] = jax.nn.sigmoid(p[:, :H]) * jnp.tanh(p[:, 2 * H:])


def _precompute(femb, wcat, bcat):
    tn = 2000
    return pl.pallas_call(
        _pre_body,
        grid=(N_NODES // tn,),
        in_specs=[
            pl.BlockSpec((tn, H), lambda i: (i, 0)),
            pl.BlockSpec((H, 3 * H), lambda i: (0, 0)),
            pl.BlockSpec((1, 3 * H), lambda i: (0, 0)),
        ],
        out_specs=[
            pl.BlockSpec((tn, 3 * H), lambda i: (i, 0)),
            pl.BlockSpec((tn, H), lambda i: (i, 0)),
        ],
        out_shape=[
            jax.ShapeDtypeStruct((N_NODES, 3 * H), _F32),
            jax.ShapeDtypeStruct((N_NODES, H), _F32),
        ],
    )(femb, wcat, bcat)


_TM = 2000


def _depth_body(xcat_ref, hn_ref, wzb_ref, ur_ref, bur_ref, whb_ref, hout_ref):
    i = pl.program_id(0)
    xz = xcat_ref[:, :H]
    xr = xcat_ref[:, H:2 * H]
    xh = xcat_ref[:, 2 * H:]
    h0 = hn_ref[0]
    h1 = hn_ref[1]
    h2 = hn_ref[2]
    h3 = hn_ref[3]
    sum_h = (h0 + h1) + (h2 + h3)
    z = jax.nn.sigmoid(xz + jnp.dot(sum_h, wzb_ref[...],
                                    preferred_element_type=_F32))
    ur = ur_ref[...]
    bur = bur_ref[...]
    sg = jnp.zeros_like(sum_h)
    for hk in (h0, h1, h2, h3):
        rk = jax.nn.sigmoid(
            xr + jnp.dot(hk, ur, preferred_element_type=_F32) + bur)
        sg = sg + rk * hk
    pre = jnp.tanh(xh + jnp.dot(sg, whb_ref[...],
                                preferred_element_type=_F32))
    hnew = sum_h + z * (pre - sum_h)
    row = lax.broadcasted_iota(_I32, hnew.shape, 0) + i * _TM
    hout_ref[...] = jnp.where(row == 0, 0.0, hnew)


def _depth(xcat, hn, wzb, ur, bur, whb):
    return pl.pallas_call(
        _depth_body,
        grid=(M_MSG // _TM,),
        in_specs=[
            pl.BlockSpec((_TM, 3 * H), lambda i: (i, 0)),
            pl.BlockSpec((K_NEI, _TM, H), lambda i: (0, i, 0)),
            pl.BlockSpec((H, H), lambda i: (0, 0)),
            pl.BlockSpec((H, H), lambda i: (0, 0)),
            pl.BlockSpec((1, H), lambda i: (0, 0)),
            pl.BlockSpec((H, H), lambda i: (0, 0)),
        ],
        out_specs=pl.BlockSpec((_TM, H), lambda i: (i, 0)),
        out_shape=jax.ShapeDtypeStruct((M_MSG, H), _F32),
    )(xcat, hn, wzb, ur, bur, whb)


def _out_body(mess_s_ref, femb_s_ref, wot_ref, wob_ref, bo_ref, tree_ref):
    nsum = (mess_s_ref[0] + mess_s_ref[1]) + (mess_s_ref[2] + mess_s_ref[3])
    acc = jnp.dot(femb_s_ref[...], wot_ref[...], preferred_element_type=_F32)
    acc = acc + jnp.dot(nsum, wob_ref[...], preferred_element_type=_F32)
    tree_ref[...] = jax.nn.relu(acc + bo_ref[...])


def _out_stage(mess_s, femb_s, wot, wob, bo):
    return pl.pallas_call(
        _out_body,
        out_shape=jax.ShapeDtypeStruct((B_TREE, H), _F32),
    )(mess_s, femb_s, wot, wob, bo)


def kernel(fnode, fmess, node_graph, mess_graph, scope, embedding,
           W_z, b_z, W_r, U_r, b_Ur, W_h, b_h, W_o, b_o):
    fnode = fnode.astype(_I32)
    fmess = fmess.astype(_I32)

    mgt = mess_graph.T.reshape(-1)
    sidx = scope[:, 0]
    ngf = jnp.take(node_graph, sidx, axis=0).T.reshape(-1)

    wcat = jnp.concatenate([W_z[:H], W_r, W_h[:H]], axis=1)
    bcat = jnp.concatenate(
        [b_z, jnp.zeros((H,), _F32), b_h]).reshape(1, 3 * H)

    femb = _make_row_gather(N_NODES, H, 400, 25, 800)(embedding, fnode)
    pnode, h1node = _precompute(femb, wcat, bcat)
    xcat, h = _make_xcat_gather(40)(pnode, h1node, fmess)

    wzb = W_z[H:]
    whb = W_h[H:]
    bur = b_Ur.reshape(1, H)
    nei_gather = _make_row_gather(K_NEI * M_MSG, H, 400, _NW, M_MSG)
    for _ in range(2):
        hn = nei_gather(h, mgt).reshape(K_NEI, M_MSG, H)
        h = _depth(xcat, hn, wzb, U_r, bur, whb)
    messages = h

    mess_s, femb_s = _make_final_gather()(messages, femb, ngf, sidx)
    tree_vecs = _out_stage(mess_s.reshape(K_NEI, B_TREE, H), femb_s,
                           W_o[:H], W_o[H:], b_o.reshape(1, H))
    return (tree_vecs, messages)


# speedup vs baseline: 7.6981x; 1.0635x over previous
import functools

import jax
import jax.numpy as jnp
from jax import lax
from jax.experimental import pallas as pl
from jax.experimental.pallas import tpu as pltpu
from jax.experimental.pallas import tpu_sc as plsc

H = 128
N_NODES = 10000
M_MSG = 160000
K_NEI = 4
B_TREE = 256
P_WORDS = 2 * H

_NC = 2
_NS = 16
_NW = _NC * _NS

_F32 = jnp.float32
_I32 = jnp.int32


def _wid():
    return lax.axis_index("s") * _NC + lax.axis_index("c")


def _sc_mesh():
    return plsc.VectorSubcoreMesh(core_axis_name="c", subcore_axis_name="s")


def _rnd_bf16_top(x):
    b = lax.bitcast_convert_type(x, _I32)
    return b + 0x7FFF + ((b >> 16) & 1)


def _pack_cols(x):
    w = x.shape[1] // 2
    lo = _rnd_bf16_top(x[:, :w])
    hi = _rnd_bf16_top(x[:, w:])
    return ((lo >> 16) & 0xFFFF) | (hi & jnp.int32(-65536))


def _unpack_cols(w):
    lo = lax.bitcast_convert_type(w << 16, _F32)
    hi = lax.bitcast_convert_type(w & jnp.int32(-65536), _F32)
    return jnp.concatenate([lo, hi], axis=1)


def _make_row_gather(total, d, chunk, active, dtype=_F32):
    per_w = total // active
    n_chunks = per_w // chunk
    assert per_w * active == total and n_chunks * chunk == per_w
    assert chunk % 8 == 0 and per_w % 8 == 0

    @functools.partial(
        pl.kernel,
        mesh=_sc_mesh(),
        out_type=jax.ShapeDtypeStruct((total, d), dtype),
        scratch_types=[
            pltpu.VMEM((per_w,), _I32),
            pltpu.VMEM((chunk, d), dtype),
            pltpu.VMEM((chunk, d), dtype),
            pltpu.SemaphoreType.DMA,
            pltpu.SemaphoreType.DMA,
        ],
    )
    def gather_k(table_hbm, idx_hbm, out_hbm, idx_v, rows_a, rows_b,
                 sem_a, sem_b):
        w = _wid()
        row_bufs = (rows_a, rows_b)
        sems = (sem_a, sem_b)

        @pl.when(w < active)
        def _():
            pltpu.sync_copy(idx_hbm.at[pl.ds(w * per_w, per_w)], idx_v)

            def issue(j, b):
                pltpu.make_async_copy(
                    table_hbm.at[idx_v.at[pl.ds(j * chunk, chunk)]],
                    row_bufs[b], sems[b]).start()

            issue(0, 0)

            def outer(j0, carry):
                for b in range(2):
                    j = 2 * j0 + b

                    @pl.when(j < n_chunks)
                    def _():
                        @pl.when(j + 1 < n_chunks)
                        def _():
                            issue(j + 1, 1 - b)

                        pltpu.make_async_copy(
                            table_hbm.at[idx_v.at[pl.ds(j * chunk, chunk)]],
                            row_bufs[b], sems[b]).wait()
                        base = w * per_w + j * chunk
                        pltpu.sync_copy(row_bufs[b],
                                        out_hbm.at[pl.ds(base, chunk)])

                return carry

            lax.fori_loop(0, (n_chunks + 1) // 2, outer, 0)

    return gather_k


def _make_xcat_gather(chunk):
    per_w = M_MSG // _NW
    n_chunks = per_w // chunk
    assert n_chunks * chunk == per_w and chunk % 8 == 0

    @functools.partial(
        pl.kernel,
        mesh=_sc_mesh(),
        out_type=(
            jax.ShapeDtypeStruct((M_MSG, P_WORDS), _I32),
            jax.ShapeDtypeStruct((M_MSG, H), _F32),
        ),
        scratch_types=[
            pltpu.VMEM((per_w,), _I32),
            pltpu.VMEM((chunk, P_WORDS), _I32),
            pltpu.VMEM((chunk, P_WORDS), _I32),
            pltpu.VMEM((chunk, H), _F32),
            pltpu.VMEM((chunk, H), _F32),
            pltpu.SemaphoreType.DMA,
            pltpu.SemaphoreType.DMA,
        ],
    )
    def xcat_k(pnode_hbm, h1node_hbm, fmess_hbm, xcat_hbm, h1_hbm,
               idx_v, p_a, p_b, h_a, h_b, sem_a, sem_b):
        w = _wid()
        p_bufs = (p_a, p_b)
        h_bufs = (h_a, h_b)
        sems = (sem_a, sem_b)

        pltpu.sync_copy(fmess_hbm.at[pl.ds(w * per_w, per_w)], idx_v)

        def issue(j, b):
            isl = idx_v.at[pl.ds(j * chunk, chunk)]
            pltpu.make_async_copy(pnode_hbm.at[isl], p_bufs[b],
                                  sems[b]).start()
            pltpu.make_async_copy(h1node_hbm.at[isl], h_bufs[b],
                                  sems[b]).start()

        issue(0, 0)

        def outer(j0, carry):
            for b in range(2):
                j = 2 * j0 + b

                @pl.when(j < n_chunks)
                def _():
                    @pl.when(j + 1 < n_chunks)
                    def _():
                        issue(j + 1, 1 - b)

                    isl = idx_v.at[pl.ds(j * chunk, chunk)]
                    pltpu.make_async_copy(pnode_hbm.at[isl], p_bufs[b],
                                          sems[b]).wait()
                    pltpu.make_async_copy(h1node_hbm.at[isl], h_bufs[b],
                                          sems[b]).wait()

                    @pl.when(jnp.logical_and(w == 0, j == 0))
                    def _():
                        for c in range(H // 16):
                            h_bufs[b][0, pl.ds(c * 16, 16)] = jnp.zeros(
                                (16,), _F32)

                    base = w * per_w + j * chunk
                    pltpu.sync_copy(p_bufs[b], xcat_hbm.at[pl.ds(base, chunk)])
                    pltpu.sync_copy(h_bufs[b], h1_hbm.at[pl.ds(base, chunk)])

            return carry

        lax.fori_loop(0, (n_chunks + 1) // 2, outer, 0)

    return xcat_k


def _make_final_gather():
    mb = (K_NEI * B_TREE) // _NW
    fb = B_TREE // _NS

    @functools.partial(
        pl.kernel,
        mesh=_sc_mesh(),
        out_type=(
            jax.ShapeDtypeStruct((K_NEI * B_TREE, H), _F32),
            jax.ShapeDtypeStruct((B_TREE, H), _F32),
        ),
        scratch_types=[
            pltpu.VMEM((mb,), _I32),
            pltpu.VMEM((mb, H), _F32),
            pltpu.VMEM((fb,), _I32),
            pltpu.VMEM((fb, H), _F32),
            pltpu.SemaphoreType.DMA,
        ],
    )
    def final_k(mess_hbm, femb_hbm, ngf_hbm, sidx_hbm, mess_s_hbm, femb_s_hbm,
                i1_v, r1_v, i2_v, r2_v, sem):
        w = _wid()
        base = w * mb
        pltpu.sync_copy(ngf_hbm.at[pl.ds(base, mb)], i1_v)
        pltpu.async_copy(mess_hbm.at[i1_v], r1_v, sem).wait()
        pltpu.sync_copy(r1_v, mess_s_hbm.at[pl.ds(base, mb)])

        @pl.when(w < _NS)
        def _():
            fbase = w * fb
            pltpu.sync_copy(sidx_hbm.at[pl.ds(fbase, fb)], i2_v)
            pltpu.async_copy(femb_hbm.at[i2_v], r2_v, sem).wait()
            pltpu.sync_copy(r2_v, femb_s_hbm.at[pl.ds(fbase, fb)])

    return final_k


def _pre_body(femb_ref, wcat_ref, bcat_ref, pnode_ref, h1n_ref):
    e = femb_ref[...]
    p = jnp.dot(e, wcat_ref[...], preferred_element_type=_F32) + bcat_ref[...]
    pnode_ref[:, :3 * H // 2] = _pack_cols(p)
    pnode_ref[:, 3 * H // 2:] = jnp.zeros(
        (e.shape[0], P_WORDS - 3 * H // 2), _I32)
    h1n_ref[...] = jax.nn.sigmoid(p[:, :H]) * jnp.tanh(p[:, 2 * H:])


def _precompute(femb, wcat, bcat):
    tn = 2000
    return pl.pallas_call(
        _pre_body,
        grid=(N_NODES // tn,),
        in_specs=[
            pl.BlockSpec((tn, H), lambda i: (i, 0)),
            pl.BlockSpec((H, 3 * H), lambda i: (0, 0)),
            pl.BlockSpec((1, 3 * H), lambda i: (0, 0)),
        ],
        out_specs=[
            pl.BlockSpec((tn, P_WORDS), lambda i: (i, 0)),
            pl.BlockSpec((tn, H), lambda i: (i, 0)),
        ],
        out_shape=[
            jax.ShapeDtypeStruct((N_NODES, P_WORDS), _I32),
            jax.ShapeDtypeStruct((N_NODES, H), _F32),
        ],
    )(femb, wcat, bcat)


_TM = 2000


def _depth_body(xcat_ref, hn_ref, wzb_ref, ur_ref, bur_ref, whb_ref,
                hout_ref):
    i = pl.program_id(0)
    x = _unpack_cols(xcat_ref[:, :3 * H // 2])
    xz = x[:, :H]
    xr = x[:, H:2 * H]
    xh = x[:, 2 * H:]
    h0 = hn_ref[0]
    h1 = hn_ref[1]
    h2 = hn_ref[2]
    h3 = hn_ref[3]
    sum_h = (h0 + h1) + (h2 + h3)
    z = jax.nn.sigmoid(xz + jnp.dot(sum_h, wzb_ref[...],
                                    preferred_element_type=_F32))
    ur = ur_ref[...]
    bur = bur_ref[...]
    sg = jnp.zeros_like(sum_h)
    for hk in (h0, h1, h2, h3):
        rk = jax.nn.sigmoid(
            xr + jnp.dot(hk, ur, preferred_element_type=_F32) + bur)
        sg = sg + rk * hk
    pre = jnp.tanh(xh + jnp.dot(sg, whb_ref[...],
                                preferred_element_type=_F32))
    hnew = sum_h + z * (pre - sum_h)
    row = lax.broadcasted_iota(_I32, hnew.shape, 0) + i * _TM
    hout_ref[...] = jnp.where(row == 0, 0.0, hnew)


def _depth(xcat, hn, wzb, ur, bur, whb):
    return pl.pallas_call(
        _depth_body,
        grid=(M_MSG // _TM,),
        in_specs=[
            pl.BlockSpec((_TM, P_WORDS), lambda i: (i, 0)),
            pl.BlockSpec((K_NEI, _TM, H), lambda i: (0, i, 0)),
            pl.BlockSpec((H, H), lambda i: (0, 0)),
            pl.BlockSpec((H, H), lambda i: (0, 0)),
            pl.BlockSpec((1, H), lambda i: (0, 0)),
            pl.BlockSpec((H, H), lambda i: (0, 0)),
        ],
        out_specs=pl.BlockSpec((_TM, H), lambda i: (i, 0)),
        out_shape=jax.ShapeDtypeStruct((M_MSG, H), _F32),
    )(xcat, hn, wzb, ur, bur, whb)


def _out_body(mess_s_ref, femb_s_ref, wot_ref, wob_ref, bo_ref, tree_ref):
    nsum = (mess_s_ref[0] + mess_s_ref[1]) + (mess_s_ref[2] + mess_s_ref[3])
    acc = jnp.dot(femb_s_ref[...], wot_ref[...], preferred_element_type=_F32)
    acc = acc + jnp.dot(nsum, wob_ref[...], preferred_element_type=_F32)
    tree_ref[...] = jax.nn.relu(acc + bo_ref[...])


def _out_stage(mess_s, femb_s, wot, wob, bo):
    return pl.pallas_call(
        _out_body,
        out_shape=jax.ShapeDtypeStruct((B_TREE, H), _F32),
    )(mess_s, femb_s, wot, wob, bo)


def kernel(fnode, fmess, node_graph, mess_graph, scope, embedding,
           W_z, b_z, W_r, U_r, b_Ur, W_h, b_h, W_o, b_o):
    fnode = fnode.astype(_I32)
    fmess = fmess.astype(_I32)

    mgt = mess_graph.T.reshape(-1)
    sidx = scope[:, 0]
    ngf = jnp.take(node_graph, sidx, axis=0).T.reshape(-1)

    wcat = jnp.concatenate([W_z[:H], W_r, W_h[:H]], axis=1)
    bcat = jnp.concatenate(
        [b_z, jnp.zeros((H,), _F32), b_h]).reshape(1, 3 * H)

    femb = _make_row_gather(N_NODES, H, 400, 25)(embedding, fnode)
    pnode, h1node = _precompute(femb, wcat, bcat)
    xcat, h = _make_xcat_gather(40)(pnode, h1node, fmess)

    wzb = W_z[H:]
    whb = W_h[H:]
    bur = b_Ur.reshape(1, H)
    nei_gather = _make_row_gather(K_NEI * M_MSG, H, 400, _NW)
    for _ in range(2):
        hn = nei_gather(h, mgt).reshape(K_NEI, M_MSG, H)
        h = _depth(xcat, hn, wzb, U_r, bur, whb)
    messages = h

    mess_s, femb_s = _make_final_gather()(messages, femb, ngf, sidx)
    tree_vecs = _out_stage(mess_s.reshape(K_NEI, B_TREE, H), femb_s,
                           W_o[:H], W_o[H:], b_o.reshape(1, H))
    return (tree_vecs, messages)


# speedup vs baseline: 7.8824x; 1.0239x over previous
import functools

import jax
import jax.numpy as jnp
from jax import lax
from jax.experimental import pallas as pl
from jax.experimental.pallas import tpu as pltpu
from jax.experimental.pallas import tpu_sc as plsc

H = 128
N_NODES = 10000
M_MSG = 160000
K_NEI = 4
B_TREE = 256
P_WORDS = 2 * H

_NC = 2
_NS = 16
_NW = _NC * _NS

_F32 = jnp.float32
_I32 = jnp.int32


def _wid():
    return lax.axis_index("s") * _NC + lax.axis_index("c")


def _sc_mesh():
    return plsc.VectorSubcoreMesh(core_axis_name="c", subcore_axis_name="s")


def _rnd_bf16_top(x):
    b = lax.bitcast_convert_type(x, _I32)
    return b + 0x7FFF + ((b >> 16) & 1)


def _pack_cols(x):
    w = x.shape[1] // 2
    lo = _rnd_bf16_top(x[:, :w])
    hi = _rnd_bf16_top(x[:, w:])
    return ((lo >> 16) & 0xFFFF) | (hi & jnp.int32(-65536))


def _unpack_cols(w):
    lo = lax.bitcast_convert_type(w << 16, _F32)
    hi = lax.bitcast_convert_type(w & jnp.int32(-65536), _F32)
    return jnp.concatenate([lo, hi], axis=1)


def _make_row_gather(total, d, chunk, active, dtype=_F32):
    per_w = total // active
    n_chunks = per_w // chunk
    assert per_w * active == total and n_chunks * chunk == per_w
    assert chunk % 8 == 0 and per_w % 8 == 0

    @functools.partial(
        pl.kernel,
        mesh=_sc_mesh(),
        out_type=jax.ShapeDtypeStruct((total, d), dtype),
        scratch_types=[
            pltpu.VMEM((per_w,), _I32),
            pltpu.VMEM((chunk, d), dtype),
            pltpu.VMEM((chunk, d), dtype),
            pltpu.SemaphoreType.DMA,
            pltpu.SemaphoreType.DMA,
        ],
    )
    def gather_k(table_hbm, idx_hbm, out_hbm, idx_v, rows_a, rows_b,
                 sem_a, sem_b):
        w = _wid()
        row_bufs = (rows_a, rows_b)
        sems = (sem_a, sem_b)

        @pl.when(w < active)
        def _():
            pltpu.sync_copy(idx_hbm.at[pl.ds(w * per_w, per_w)], idx_v)

            def issue(j, b):
                pltpu.make_async_copy(
                    table_hbm.at[idx_v.at[pl.ds(j * chunk, chunk)]],
                    row_bufs[b], sems[b]).start()

            issue(0, 0)

            def outer(j0, carry):
                for b in range(2):
                    j = 2 * j0 + b

                    @pl.when(j < n_chunks)
                    def _():
                        @pl.when(j + 1 < n_chunks)
                        def _():
                            issue(j + 1, 1 - b)

                        pltpu.make_async_copy(
                            table_hbm.at[idx_v.at[pl.ds(j * chunk, chunk)]],
                            row_bufs[b], sems[b]).wait()
                        base = w * per_w + j * chunk
                        pltpu.sync_copy(row_bufs[b],
                                        out_hbm.at[pl.ds(base, chunk)])

                return carry

            lax.fori_loop(0, (n_chunks + 1) // 2, outer, 0)

    return gather_k


def _make_xcat_gather(chunk):
    per_w = M_MSG // _NW
    n_chunks = per_w // chunk
    assert n_chunks * chunk == per_w and chunk % 8 == 0

    @functools.partial(
        pl.kernel,
        mesh=_sc_mesh(),
        out_type=(
            jax.ShapeDtypeStruct((M_MSG, P_WORDS), _I32),
            jax.ShapeDtypeStruct((M_MSG, H), _F32),
        ),
        scratch_types=[
            pltpu.VMEM((per_w,), _I32),
            pltpu.VMEM((chunk, P_WORDS), _I32),
            pltpu.VMEM((chunk, P_WORDS), _I32),
            pltpu.VMEM((chunk, H), _F32),
            pltpu.VMEM((chunk, H), _F32),
            pltpu.SemaphoreType.DMA,
            pltpu.SemaphoreType.DMA,
        ],
    )
    def xcat_k(pnode_hbm, h1node_hbm, fmess_hbm, xcat_hbm, h1_hbm,
               idx_v, p_a, p_b, h_a, h_b, sem_a, sem_b):
        w = _wid()
        p_bufs = (p_a, p_b)
        h_bufs = (h_a, h_b)
        sems = (sem_a, sem_b)

        pltpu.sync_copy(fmess_hbm.at[pl.ds(w * per_w, per_w)], idx_v)

        def issue(j, b):
            isl = idx_v.at[pl.ds(j * chunk, chunk)]
            pltpu.make_async_copy(pnode_hbm.at[isl], p_bufs[b],
                                  sems[b]).start()
            pltpu.make_async_copy(h1node_hbm.at[isl], h_bufs[b],
                                  sems[b]).start()

        issue(0, 0)

        def outer(j0, carry):
            for b in range(2):
                j = 2 * j0 + b

                @pl.when(j < n_chunks)
                def _():
                    @pl.when(j + 1 < n_chunks)
                    def _():
                        issue(j + 1, 1 - b)

                    isl = idx_v.at[pl.ds(j * chunk, chunk)]
                    pltpu.make_async_copy(pnode_hbm.at[isl], p_bufs[b],
                                          sems[b]).wait()
                    pltpu.make_async_copy(h1node_hbm.at[isl], h_bufs[b],
                                          sems[b]).wait()

                    @pl.when(jnp.logical_and(w == 0, j == 0))
                    def _():
                        for c in range(H // 16):
                            h_bufs[b][0, pl.ds(c * 16, 16)] = jnp.zeros(
                                (16,), _F32)

                    base = w * per_w + j * chunk
                    pltpu.sync_copy(p_bufs[b], xcat_hbm.at[pl.ds(base, chunk)])
                    pltpu.sync_copy(h_bufs[b], h1_hbm.at[pl.ds(base, chunk)])

            return carry

        lax.fori_loop(0, (n_chunks + 1) // 2, outer, 0)

    return xcat_k


def _make_final_gather():
    mb = (K_NEI * B_TREE) // _NW
    fb = B_TREE // _NS

    @functools.partial(
        pl.kernel,
        mesh=_sc_mesh(),
        out_type=(
            jax.ShapeDtypeStruct((K_NEI * B_TREE, H), _F32),
            jax.ShapeDtypeStruct((B_TREE, H), _F32),
        ),
        scratch_types=[
            pltpu.VMEM((mb,), _I32),
            pltpu.VMEM((mb, H), _F32),
            pltpu.VMEM((fb,), _I32),
            pltpu.VMEM((fb, H), _F32),
            pltpu.SemaphoreType.DMA,
        ],
    )
    def final_k(mess_hbm, femb_hbm, ngf_hbm, sidx_hbm, mess_s_hbm, femb_s_hbm,
                i1_v, r1_v, i2_v, r2_v, sem):
        w = _wid()
        base = w * mb
        pltpu.sync_copy(ngf_hbm.at[pl.ds(base, mb)], i1_v)
        pltpu.async_copy(mess_hbm.at[i1_v], r1_v, sem).wait()
        pltpu.sync_copy(r1_v, mess_s_hbm.at[pl.ds(base, mb)])

        @pl.when(w < _NS)
        def _():
            fbase = w * fb
            pltpu.sync_copy(sidx_hbm.at[pl.ds(fbase, fb)], i2_v)
            pltpu.async_copy(femb_hbm.at[i2_v], r2_v, sem).wait()
            pltpu.sync_copy(r2_v, femb_s_hbm.at[pl.ds(fbase, fb)])

    return final_k


def _pre_body(femb_ref, wcat_ref, bcat_ref, pnode_ref, h1n_ref):
    e = femb_ref[...]
    p = jnp.dot(e, wcat_ref[...], preferred_element_type=_F32) + bcat_ref[...]
    pnode_ref[:, :3 * H // 2] = _pack_cols(p)
    pnode_ref[:, 3 * H // 2:] = jnp.zeros(
        (e.shape[0], P_WORDS - 3 * H // 2), _I32)
    h1n_ref[...] = jax.nn.sigmoid(p[:, :H]) * jnp.tanh(p[:, 2 * H:])


def _precompute(femb, wcat, bcat):
    tn = 2000
    return pl.pallas_call(
        _pre_body,
        grid=(N_NODES // tn,),
        in_specs=[
            pl.BlockSpec((tn, H), lambda i: (i, 0)),
            pl.BlockSpec((H, 3 * H), lambda i: (0, 0)),
            pl.BlockSpec((1, 3 * H), lambda i: (0, 0)),
        ],
        out_specs=[
            pl.BlockSpec((tn, P_WORDS), lambda i: (i, 0)),
            pl.BlockSpec((tn, H), lambda i: (i, 0)),
        ],
        out_shape=[
            jax.ShapeDtypeStruct((N_NODES, P_WORDS), _I32),
            jax.ShapeDtypeStruct((N_NODES, H), _F32),
        ],
    )(femb, wcat, bcat)


_TM = 2000
_M_HALF = M_MSG // 2
_HB = _M_HALF // _TM


def _make_depth_half_body(off, with_prev):
    def body(*refs):
        if with_prev:
            (_prev_ref, xcat_ref, hn_ref, wzb_ref, ur_ref, bur_ref,
             whb_ref, hout_ref) = refs
        else:
            (xcat_ref, hn_ref, wzb_ref, ur_ref, bur_ref, whb_ref,
             hout_ref) = refs
        i = pl.program_id(0)
        x = _unpack_cols(xcat_ref[:, :3 * H // 2])
        xz = x[:, :H]
        xr = x[:, H:2 * H]
        xh = x[:, 2 * H:]
        h0 = hn_ref[0]
        h1 = hn_ref[1]
        h2 = hn_ref[2]
        h3 = hn_ref[3]
        sum_h = (h0 + h1) + (h2 + h3)
        z = jax.nn.sigmoid(xz + jnp.dot(sum_h, wzb_ref[...],
                                        preferred_element_type=_F32))
        ur = ur_ref[...]
        bur = bur_ref[...]
        sg = jnp.zeros_like(sum_h)
        for hk in (h0, h1, h2, h3):
            rk = jax.nn.sigmoid(
                xr + jnp.dot(hk, ur, preferred_element_type=_F32) + bur)
            sg = sg + rk * hk
        pre = jnp.tanh(xh + jnp.dot(sg, whb_ref[...],
                                    preferred_element_type=_F32))
        hnew = sum_h + z * (pre - sum_h)
        row = lax.broadcasted_iota(_I32, hnew.shape, 0) + (i + off) * _TM
        hout_ref[...] = jnp.where(row == 0, 0.0, hnew)

    return body


def _depth_half(xcat, hn, wzb, ur, bur, whb, prev=None):
    with_prev = prev is not None
    off = _HB if with_prev else 0
    half_specs = [
        pl.BlockSpec((_TM, P_WORDS), lambda i: (i + off, 0)),
        pl.BlockSpec((K_NEI, _TM, H), lambda i: (0, i, 0)),
        pl.BlockSpec((H, H), lambda i: (0, 0)),
        pl.BlockSpec((H, H), lambda i: (0, 0)),
        pl.BlockSpec((1, H), lambda i: (0, 0)),
        pl.BlockSpec((H, H), lambda i: (0, 0)),
    ]
    args = (xcat, hn, wzb, ur, bur, whb)
    if with_prev:
        half_specs = [pl.BlockSpec((8, H), lambda i: (0, 0))] + half_specs
        args = (prev,) + args
    return pl.pallas_call(
        _make_depth_half_body(off, with_prev),
        grid=(_HB,),
        in_specs=half_specs,
        out_specs=pl.BlockSpec((_TM, H), lambda i: (i + off, 0)),
        out_shape=jax.ShapeDtypeStruct((M_MSG, H), _F32),
        input_output_aliases={0: 0} if with_prev else {},
    )(*args)


def _out_body(mess_s_ref, femb_s_ref, wot_ref, wob_ref, bo_ref, tree_ref):
    nsum = (mess_s_ref[0] + mess_s_ref[1]) + (mess_s_ref[2] + mess_s_ref[3])
    acc = jnp.dot(femb_s_ref[...], wot_ref[...], preferred_element_type=_F32)
    acc = acc + jnp.dot(nsum, wob_ref[...], preferred_element_type=_F32)
    tree_ref[...] = jax.nn.relu(acc + bo_ref[...])


def _out_stage(mess_s, femb_s, wot, wob, bo):
    return pl.pallas_call(
        _out_body,
        out_shape=jax.ShapeDtypeStruct((B_TREE, H), _F32),
    )(mess_s, femb_s, wot, wob, bo)


def kernel(fnode, fmess, node_graph, mess_graph, scope, embedding,
           W_z, b_z, W_r, U_r, b_Ur, W_h, b_h, W_o, b_o):
    fnode = fnode.astype(_I32)
    fmess = fmess.astype(_I32)

    mg2 = mess_graph.T
    mgt_lo = mg2[:, :_M_HALF].reshape(-1)
    mgt_hi = mg2[:, _M_HALF:].reshape(-1)
    sidx = scope[:, 0]
    ngf = jnp.take(node_graph, sidx, axis=0).T.reshape(-1)

    wcat = jnp.concatenate([W_z[:H], W_r, W_h[:H]], axis=1)
    bcat = jnp.concatenate(
        [b_z, jnp.zeros((H,), _F32), b_h]).reshape(1, 3 * H)

    femb = _make_row_gather(N_NODES, H, 400, 25)(embedding, fnode)
    pnode, h1node = _precompute(femb, wcat, bcat)
    xcat, h = _make_xcat_gather(40)(pnode, h1node, fmess)

    wzb = W_z[H:]
    whb = W_h[H:]
    bur = b_Ur.reshape(1, H)
    nei_gather = _make_row_gather(K_NEI * _M_HALF, H, 400, _NW)
    for _ in range(2):
        hn_lo = nei_gather(h, mgt_lo).reshape(K_NEI, _M_HALF, H)
        hn_hi = nei_gather(h, mgt_hi).reshape(K_NEI, _M_HALF, H)
        h_lo = _depth_half(xcat, hn_lo, wzb, U_r, bur, whb)
        h = _depth_half(xcat, hn_hi, wzb, U_r, bur, whb, prev=h_lo)
    messages = h

    mess_s, femb_s = _make_final_gather()(messages, femb, ngf, sidx)
    tree_vecs = _out_stage(mess_s.reshape(K_NEI, B_TREE, H), femb_s,
                           W_o[:H], W_o[H:], b_o.reshape(1, H))
    return (tree_vecs, messages)


# speedup vs baseline: 7.9820x; 1.0126x over previous
import functools

import jax
import jax.numpy as jnp
from jax import lax
from jax.experimental import pallas as pl
from jax.experimental.pallas import tpu as pltpu
from jax.experimental.pallas import tpu_sc as plsc

H = 128
N_NODES = 10000
M_MSG = 160000
K_NEI = 4
B_TREE = 256
P_WORDS = 2 * H

_NC = 2
_NS = 16
_NW = _NC * _NS

_F32 = jnp.float32
_I32 = jnp.int32


def _wid():
    return lax.axis_index("s") * _NC + lax.axis_index("c")


def _sc_mesh():
    return plsc.VectorSubcoreMesh(core_axis_name="c", subcore_axis_name="s")


def _rnd_bf16_top(x):
    b = lax.bitcast_convert_type(x, _I32)
    return b + 0x7FFF + ((b >> 16) & 1)


def _pack_cols(x):
    w = x.shape[1] // 2
    lo = _rnd_bf16_top(x[:, :w])
    hi = _rnd_bf16_top(x[:, w:])
    return ((lo >> 16) & 0xFFFF) | (hi & jnp.int32(-65536))


def _unpack_cols(w):
    lo = lax.bitcast_convert_type(w << 16, _F32)
    hi = lax.bitcast_convert_type(w & jnp.int32(-65536), _F32)
    return jnp.concatenate([lo, hi], axis=1)


def _make_row_gather(total, d, chunk, active, dtype=_F32):
    per_w = total // active
    n_chunks = per_w // chunk
    assert per_w * active == total and n_chunks * chunk == per_w
    assert chunk % 8 == 0 and per_w % 8 == 0

    @functools.partial(
        pl.kernel,
        mesh=_sc_mesh(),
        out_type=jax.ShapeDtypeStruct((total, d), dtype),
        scratch_types=[
            pltpu.VMEM((per_w,), _I32),
            pltpu.VMEM((chunk, d), dtype),
            pltpu.VMEM((chunk, d), dtype),
            pltpu.SemaphoreType.DMA,
            pltpu.SemaphoreType.DMA,
        ],
    )
    def gather_k(table_hbm, idx_hbm, out_hbm, idx_v, rows_a, rows_b,
                 sem_a, sem_b):
        w = _wid()
        row_bufs = (rows_a, rows_b)
        sems = (sem_a, sem_b)

        @pl.when(w < active)
        def _():
            pltpu.sync_copy(idx_hbm.at[pl.ds(w * per_w, per_w)], idx_v)

            def issue(j, b):
                pltpu.make_async_copy(
                    table_hbm.at[idx_v.at[pl.ds(j * chunk, chunk)]],
                    row_bufs[b], sems[b]).start()

            issue(0, 0)

            def outer(j0, carry):
                for b in range(2):
                    j = 2 * j0 + b

                    @pl.when(j < n_chunks)
                    def _():
                        @pl.when(j + 1 < n_chunks)
                        def _():
                            issue(j + 1, 1 - b)

                        pltpu.make_async_copy(
                            table_hbm.at[idx_v.at[pl.ds(j * chunk, chunk)]],
                            row_bufs[b], sems[b]).wait()
                        base = w * per_w + j * chunk
                        pltpu.sync_copy(row_bufs[b],
                                        out_hbm.at[pl.ds(base, chunk)])

                return carry

            lax.fori_loop(0, (n_chunks + 1) // 2, outer, 0)

    return gather_k


def _make_xcat_gather(chunk):
    per_w = M_MSG // _NW
    n_chunks = per_w // chunk
    assert n_chunks * chunk == per_w and chunk % 8 == 0

    @functools.partial(
        pl.kernel,
        mesh=_sc_mesh(),
        out_type=(
            jax.ShapeDtypeStruct((M_MSG, P_WORDS), _I32),
            jax.ShapeDtypeStruct((M_MSG, H), _F32),
        ),
        scratch_types=[
            pltpu.VMEM((per_w,), _I32),
            pltpu.VMEM((chunk, P_WORDS), _I32),
            pltpu.VMEM((chunk, P_WORDS), _I32),
            pltpu.VMEM((chunk, H), _F32),
            pltpu.VMEM((chunk, H), _F32),
            pltpu.SemaphoreType.DMA,
            pltpu.SemaphoreType.DMA,
        ],
    )
    def xcat_k(pnode_hbm, h1node_hbm, fmess_hbm, xcat_hbm, h1_hbm,
               idx_v, p_a, p_b, h_a, h_b, sem_a, sem_b):
        w = _wid()
        p_bufs = (p_a, p_b)
        h_bufs = (h_a, h_b)
        sems = (sem_a, sem_b)

        pltpu.sync_copy(fmess_hbm.at[pl.ds(w * per_w, per_w)], idx_v)

        def issue(j, b):
            isl = idx_v.at[pl.ds(j * chunk, chunk)]
            pltpu.make_async_copy(pnode_hbm.at[isl], p_bufs[b],
                                  sems[b]).start()
            pltpu.make_async_copy(h1node_hbm.at[isl], h_bufs[b],
                                  sems[b]).start()

        issue(0, 0)

        def outer(j0, carry):
            for b in range(2):
                j = 2 * j0 + b

                @pl.when(j < n_chunks)
                def _():
                    @pl.when(j + 1 < n_chunks)
                    def _():
                        issue(j + 1, 1 - b)

                    isl = idx_v.at[pl.ds(j * chunk, chunk)]
                    pltpu.make_async_copy(pnode_hbm.at[isl], p_bufs[b],
                                          sems[b]).wait()
                    pltpu.make_async_copy(h1node_hbm.at[isl], h_bufs[b],
                                          sems[b]).wait()

                    @pl.when(jnp.logical_and(w == 0, j == 0))
                    def _():
                        for c in range(H // 16):
                            h_bufs[b][0, pl.ds(c * 16, 16)] = jnp.zeros(
                                (16,), _F32)

                    base = w * per_w + j * chunk
                    pltpu.sync_copy(p_bufs[b], xcat_hbm.at[pl.ds(base, chunk)])
                    pltpu.sync_copy(h_bufs[b], h1_hbm.at[pl.ds(base, chunk)])

            return carry

        lax.fori_loop(0, (n_chunks + 1) // 2, outer, 0)

    return xcat_k


def _make_final_gather():
    mb = (K_NEI * B_TREE) // _NW
    fb = B_TREE // _NS

    @functools.partial(
        pl.kernel,
        mesh=_sc_mesh(),
        out_type=(
            jax.ShapeDtypeStruct((K_NEI * B_TREE, H), _F32),
            jax.ShapeDtypeStruct((B_TREE, H), _F32),
        ),
        scratch_types=[
            pltpu.VMEM((mb,), _I32),
            pltpu.VMEM((mb, H), _F32),
            pltpu.VMEM((fb,), _I32),
            pltpu.VMEM((fb, H), _F32),
            pltpu.SemaphoreType.DMA,
        ],
    )
    def final_k(mess_hbm, femb_hbm, ngf_hbm, sidx_hbm, mess_s_hbm, femb_s_hbm,
                i1_v, r1_v, i2_v, r2_v, sem):
        w = _wid()
        base = w * mb
        pltpu.sync_copy(ngf_hbm.at[pl.ds(base, mb)], i1_v)
        pltpu.async_copy(mess_hbm.at[i1_v], r1_v, sem).wait()
        pltpu.sync_copy(r1_v, mess_s_hbm.at[pl.ds(base, mb)])

        @pl.when(w < _NS)
        def _():
            fbase = w * fb
            pltpu.sync_copy(sidx_hbm.at[pl.ds(fbase, fb)], i2_v)
            pltpu.async_copy(femb_hbm.at[i2_v], r2_v, sem).wait()
            pltpu.sync_copy(r2_v, femb_s_hbm.at[pl.ds(fbase, fb)])

    return final_k


def _pre_body(femb_ref, wcat_ref, bcat_ref, pnode_ref, h1n_ref):
    e = femb_ref[...]
    p = jnp.dot(e, wcat_ref[...], preferred_element_type=_F32) + bcat_ref[...]
    pnode_ref[:, :3 * H // 2] = _pack_cols(p)
    pnode_ref[:, 3 * H // 2:] = jnp.zeros(
        (e.shape[0], P_WORDS - 3 * H // 2), _I32)
    h1n_ref[...] = jax.nn.sigmoid(p[:, :H]) * jnp.tanh(p[:, 2 * H:])


def _precompute(femb, wcat, bcat):
    tn = 2000
    return pl.pallas_call(
        _pre_body,
        grid=(N_NODES // tn,),
        in_specs=[
            pl.BlockSpec((tn, H), lambda i: (i, 0)),
            pl.BlockSpec((H, 3 * H), lambda i: (0, 0)),
            pl.BlockSpec((1, 3 * H), lambda i: (0, 0)),
        ],
        out_specs=[
            pl.BlockSpec((tn, P_WORDS), lambda i: (i, 0)),
            pl.BlockSpec((tn, H), lambda i: (i, 0)),
        ],
        out_shape=[
            jax.ShapeDtypeStruct((N_NODES, P_WORDS), _I32),
            jax.ShapeDtypeStruct((N_NODES, H), _F32),
        ],
    )(femb, wcat, bcat)


_TM = 2000
_NSPLIT = 4
_M_PART = M_MSG // _NSPLIT
_PB = _M_PART // _TM


def _make_depth_part_body(off, with_prev):
    def body(*refs):
        if with_prev:
            (_prev_ref, xcat_ref, hn_ref, wzb_ref, ur_ref, bur_ref,
             whb_ref, hout_ref) = refs
        else:
            (xcat_ref, hn_ref, wzb_ref, ur_ref, bur_ref, whb_ref,
             hout_ref) = refs
        i = pl.program_id(0)
        x = _unpack_cols(xcat_ref[:, :3 * H // 2])
        xz = x[:, :H]
        xr = x[:, H:2 * H]
        xh = x[:, 2 * H:]
        h0 = hn_ref[0]
        h1 = hn_ref[1]
        h2 = hn_ref[2]
        h3 = hn_ref[3]
        sum_h = (h0 + h1) + (h2 + h3)
        z = jax.nn.sigmoid(xz + jnp.dot(sum_h, wzb_ref[...],
                                        preferred_element_type=_F32))
        ur = ur_ref[...]
        bur = bur_ref[...]
        sg = jnp.zeros_like(sum_h)
        for hk in (h0, h1, h2, h3):
            rk = jax.nn.sigmoid(
                xr + jnp.dot(hk, ur, preferred_element_type=_F32) + bur)
            sg = sg + rk * hk
        pre = jnp.tanh(xh + jnp.dot(sg, whb_ref[...],
                                    preferred_element_type=_F32))
        hnew = sum_h + z * (pre - sum_h)
        row = lax.broadcasted_iota(_I32, hnew.shape, 0) + (i + off) * _TM
        hout_ref[...] = jnp.where(row == 0, 0.0, hnew)

    return body


def _depth_part(part, xcat, hn, wzb, ur, bur, whb, prev=None):
    with_prev = prev is not None
    off = part * _PB
    half_specs = [
        pl.BlockSpec((_TM, P_WORDS), lambda i: (i + off, 0)),
        pl.BlockSpec((K_NEI, _TM, H), lambda i: (0, i, 0)),
        pl.BlockSpec((H, H), lambda i: (0, 0)),
        pl.BlockSpec((H, H), lambda i: (0, 0)),
        pl.BlockSpec((1, H), lambda i: (0, 0)),
        pl.BlockSpec((H, H), lambda i: (0, 0)),
    ]
    args = (xcat, hn, wzb, ur, bur, whb)
    if with_prev:
        half_specs = [pl.BlockSpec((8, H), lambda i: (0, 0))] + half_specs
        args = (prev,) + args
    return pl.pallas_call(
        _make_depth_part_body(off, with_prev),
        grid=(_PB,),
        in_specs=half_specs,
        out_specs=pl.BlockSpec((_TM, H), lambda i: (i + off, 0)),
        out_shape=jax.ShapeDtypeStruct((M_MSG, H), _F32),
        input_output_aliases={0: 0} if with_prev else {},
    )(*args)


def _out_body(mess_s_ref, femb_s_ref, wot_ref, wob_ref, bo_ref, tree_ref):
    nsum = (mess_s_ref[0] + mess_s_ref[1]) + (mess_s_ref[2] + mess_s_ref[3])
    acc = jnp.dot(femb_s_ref[...], wot_ref[...], preferred_element_type=_F32)
    acc = acc + jnp.dot(nsum, wob_ref[...], preferred_element_type=_F32)
    tree_ref[...] = jax.nn.relu(acc + bo_ref[...])


def _out_stage(mess_s, femb_s, wot, wob, bo):
    return pl.pallas_call(
        _out_body,
        out_shape=jax.ShapeDtypeStruct((B_TREE, H), _F32),
    )(mess_s, femb_s, wot, wob, bo)


def kernel(fnode, fmess, node_graph, mess_graph, scope, embedding,
           W_z, b_z, W_r, U_r, b_Ur, W_h, b_h, W_o, b_o):
    fnode = fnode.astype(_I32)
    fmess = fmess.astype(_I32)

    mg2 = mess_graph.T
    mgt_parts = [mg2[:, s * _M_PART:(s + 1) * _M_PART].reshape(-1)
                 for s in range(_NSPLIT)]
    sidx = scope[:, 0]
    ngf = jnp.take(node_graph, sidx, axis=0).T.reshape(-1)

    wcat = jnp.concatenate([W_z[:H], W_r, W_h[:H]], axis=1)
    bcat = jnp.concatenate(
        [b_z, jnp.zeros((H,), _F32), b_h]).reshape(1, 3 * H)

    femb = _make_row_gather(N_NODES, H, 400, 25)(embedding, fnode)
    pnode, h1node = _precompute(femb, wcat, bcat)
    xcat, h = _make_xcat_gather(40)(pnode, h1node, fmess)

    wzb = W_z[H:]
    whb = W_h[H:]
    bur = b_Ur.reshape(1, H)
    nei_gather = _make_row_gather(K_NEI * _M_PART, H, 200, _NW)
    for _ in range(2):
        hns = [nei_gather(h, mgt_parts[s]).reshape(K_NEI, _M_PART, H)
               for s in range(_NSPLIT)]
        hp = None
        for s in range(_NSPLIT):
            hp = _depth_part(s, xcat, hns[s], wzb, U_r, bur, whb, prev=hp)
        h = hp
    messages = h

    mess_s, femb_s = _make_final_gather()(messages, femb, ngf, sidx)
    tree_vecs = _out_stage(mess_s.reshape(K_NEI, B_TREE, H), femb_s,
                           W_o[:H], W_o[H:], b_o.reshape(1, H))
    return (tree_vecs, messages)
